# Initial kernel scaffold; baseline (speedup 1.0000x reference)
import functools

import jax
import jax.numpy as jnp
from jax import lax
from jax.experimental import pallas as pl
from jax.experimental.pallas import tpu as pltpu
from jax.experimental.pallas import tpu_sc as plsc

N = 10000
NP = 10240
E = 320000
D = 128
NC = 2
NS = 16
NW = NC * NS
EPW = E // NW
K = 80
NCH = EPW // K
RPT = NP // NS

_mesh = plsc.VectorSubcoreMesh(core_axis_name="c", subcore_axis_name="s")


def _sc_agg_body(with_counts, x_hbm, src_hbm, dst_hbm, *rest):
    if with_counts:
        (sum_out, cnt_out, src_v, dst_v, rows_v, ones_v, zc_v,
         sum_sp, cnt_sp, sem) = rest
    else:
        sum_out, src_v, dst_v, rows_v, zc_v, sum_sp, sem = rest
    c = lax.axis_index("c")
    s = lax.axis_index("s")
    wid = c * NS + s

    def _zrows(i, _):
        rows_v[i // (D // 16), pl.ds((i % (D // 16)) * 16, 16)] = (
            jnp.zeros((16,), jnp.float32))
        return 0
    lax.fori_loop(0, K * (D // 16), _zrows, 0)

    def _zsp(i, _):
        pltpu.sync_copy(rows_v, sum_sp.at[pl.ds(s * RPT + i * K, K)])
        return 0
    lax.fori_loop(0, RPT // K, _zsp, 0)

    if with_counts:
        def _zc(i, _):
            zc_v[i, pl.ds(0, 16)] = jnp.zeros((16,), jnp.float32)
            ones_v[i, pl.ds(0, 16)] = jnp.ones((16,), jnp.float32)
            return 0
        lax.fori_loop(0, K, _zc, 0)

        def _zspc(i, _):
            pltpu.sync_copy(zc_v, cnt_sp.at[pl.ds(s * RPT + i * K, K)])
            return 0
        lax.fori_loop(0, RPT // K, _zspc, 0)

    pltpu.sync_copy(src_hbm.at[wid], src_v)
    pltpu.sync_copy(dst_hbm.at[wid], dst_v)

    plsc.subcore_barrier()

    def _chunk(i, _):
        pltpu.async_copy(x_hbm.at[src_v.at[i]], rows_v, sem).wait()
        pltpu.sync_copy(rows_v, sum_sp.at[dst_v.at[i]], add=True)
        if with_counts:
            pltpu.sync_copy(ones_v, cnt_sp.at[dst_v.at[i]], add=True)
        return 0
    lax.fori_loop(0, NCH, _chunk, 0)

    plsc.subcore_barrier()

    pltpu.sync_copy(sum_sp.at[pl.ds(s * RPT, RPT)],
                    sum_out.at[c, pl.ds(s * RPT, RPT)])
    if with_counts:
        pltpu.sync_copy(cnt_sp.at[pl.ds(s * RPT, RPT)],
                        cnt_out.at[c, pl.ds(s * RPT, RPT)])


_sc_agg_counts = functools.partial(
    pl.kernel,
    mesh=_mesh,
    out_type=(
        jax.ShapeDtypeStruct((NC, NP, D), jnp.float32),
        jax.ShapeDtypeStruct((NC, NP, 16), jnp.float32),
    ),
    scratch_types=[
        pltpu.VMEM((NCH, K), jnp.int32),
        pltpu.VMEM((NCH, K), jnp.int32),
        pltpu.VMEM((K, D), jnp.float32),
        pltpu.VMEM((K, 16), jnp.float32),
        pltpu.VMEM((K, 16), jnp.float32),
        pltpu.VMEM_SHARED((NP, D), jnp.float32),
        pltpu.VMEM_SHARED((NP, 16), jnp.float32),
        pltpu.SemaphoreType.DMA,
    ],
)(functools.partial(_sc_agg_body, True))

_sc_agg_nocounts = functools.partial(
    pl.kernel,
    mesh=_mesh,
    out_type=jax.ShapeDtypeStruct((NC, NP, D), jnp.float32),
    scratch_types=[
        pltpu.VMEM((NCH, K), jnp.int32),
        pltpu.VMEM((NCH, K), jnp.int32),
        pltpu.VMEM((K, D), jnp.float32),
        pltpu.VMEM_SHARED((NP, D), jnp.float32),
        pltpu.SemaphoreType.DMA,
    ],
)(functools.partial(_sc_agg_body, False))


R = 1000


def _tc_layer_body(relu, sum_ref, cnt_ref, x_ref, wlT_ref, b_ref, wrT_ref,
                   out_ref):
    ssum = sum_ref[0] + sum_ref[1]
    cnt = cnt_ref[0] + cnt_ref[1]
    inv = 1.0 / jnp.maximum(cnt[:, 0:1], 1.0)
    mean = ssum * inv
    acc = (jnp.dot(mean, wlT_ref[...], preferred_element_type=jnp.float32)
           + b_ref[...]
           + jnp.dot(x_ref[...], wrT_ref[...],
                     preferred_element_type=jnp.float32))
    out_ref[...] = jnp.maximum(acc, 0.0) if relu else acc


def _tc_layer(relu, sums, cnts, x, wlT, b, wrT):
    grid = (N // R,)
    return pl.pallas_call(
        functools.partial(_tc_layer_body, relu),
        grid=grid,
        in_specs=[
            pl.BlockSpec((NC, R, D), lambda i: (0, i, 0)),
            pl.BlockSpec((NC, R, 16), lambda i: (0, i, 0)),
            pl.BlockSpec((R, D), lambda i: (i, 0)),
            pl.BlockSpec((D, D), lambda i: (0, 0)),
            pl.BlockSpec((1, D), lambda i: (0, 0)),
            pl.BlockSpec((D, D), lambda i: (0, 0)),
        ],
        out_specs=pl.BlockSpec((R, D), lambda i: (i, 0)),
        out_shape=jax.ShapeDtypeStruct((N, D), jnp.float32),
    )(sums, cnts, x, wlT, b, wrT)


def kernel(x, edge_index, W1_l, b1_l, W1_r, W2_l, b2_l, W2_r):
    src = edge_index[0].astype(jnp.int32).reshape(NW, NCH, K)
    dst = edge_index[1].astype(jnp.int32).reshape(NW, NCH, K)

    sums1, cnts = _sc_agg_counts(x, src, dst)
    h = _tc_layer(True, sums1, cnts, x, W1_l.T, b1_l.reshape(1, D), W1_r.T)
    sums2 = _sc_agg_nocounts(h, src, dst)
    out = _tc_layer(False, sums2, cnts, h, W2_l.T, b2_l.reshape(1, D),
                    W2_r.T)
    return out


# speedup vs baseline: 7.1136x; 7.1136x over previous
import functools

import jax
import jax.numpy as jnp
from jax import lax
from jax.experimental import pallas as pl
from jax.experimental.pallas import tpu as pltpu
from jax.experimental.pallas import tpu_sc as plsc

N = 10000
NP = 10240
E = 320000
D = 128
DA = 144
NC = 2
NS = 16
NW = NC * NS
EPW = E // NW
K = 80
NCH = EPW // K
RPT = NP // NS

def _sc_agg_body(W, x_hbm, src_hbm, dst_hbm, sum_out, src_v, dst_v, rows_v,
                 sum_sp, sem):
    c = lax.axis_index("c")
    s = lax.axis_index("s")
    wid = c * NS + s

    def _zrows(i, _):
        rows_v[i // (W // 16), pl.ds((i % (W // 16)) * 16, 16)] = (
            jnp.zeros((16,), jnp.float32))
        return 0
    lax.fori_loop(0, K * (W // 16), _zrows, 0)

    def _zsp(i, _):
        pltpu.sync_copy(rows_v, sum_sp.at[pl.ds(s * RPT + i * K, K)])
        return 0
    lax.fori_loop(0, RPT // K, _zsp, 0)

    pltpu.sync_copy(src_hbm.at[wid], src_v)
    pltpu.sync_copy(dst_hbm.at[wid], dst_v)

    plsc.subcore_barrier()

    def _chunk(i, _):
        pltpu.async_copy(x_hbm.at[src_v.at[i]], rows_v, sem).wait()
        pltpu.sync_copy(rows_v, sum_sp.at[dst_v.at[i]], add=True)
        return 0
    lax.fori_loop(0, NCH, _chunk, 0)

    plsc.subcore_barrier()

    pltpu.sync_copy(sum_sp.at[pl.ds(s * RPT, RPT)],
                    sum_out.at[c, pl.ds(s * RPT, RPT)])


@functools.cache
def _make_sc_agg(W):
    mesh = plsc.VectorSubcoreMesh(core_axis_name="c", subcore_axis_name="s")
    return functools.partial(
        pl.kernel,
        mesh=mesh,
        out_type=jax.ShapeDtypeStruct((NC, NP, W), jnp.float32),
        scratch_types=[
            pltpu.VMEM((NCH, K), jnp.int32),
            pltpu.VMEM((NCH, K), jnp.int32),
            pltpu.VMEM((K, W), jnp.float32),
            pltpu.VMEM_SHARED((NP, W), jnp.float32),
            pltpu.SemaphoreType.DMA,
        ],
        compiler_params=pltpu.CompilerParams(use_tc_tiling_on_sc=False),
    )(functools.partial(_sc_agg_body, W))


R = 1000


def _tc_layer_body(relu, sum_ref, cnt_ref, x_ref, wlT_ref, b_ref, wrT_ref,
                   out_ref):
    ssum = sum_ref[0] + sum_ref[1]
    cnt = cnt_ref[0] + cnt_ref[1]
    inv = 1.0 / jnp.maximum(cnt[:, 0:1], 1.0)
    mean = ssum * inv
    acc = (jnp.dot(mean, wlT_ref[...], preferred_element_type=jnp.float32)
           + b_ref[...]
           + jnp.dot(x_ref[...], wrT_ref[...],
                     preferred_element_type=jnp.float32))
    out_ref[...] = jnp.maximum(acc, 0.0) if relu else acc


def _tc_layer(relu, sums, cnts, x, wlT, b, wrT):
    grid = (N // R,)
    return pl.pallas_call(
        functools.partial(_tc_layer_body, relu),
        grid=grid,
        in_specs=[
            pl.BlockSpec((NC, R, D), lambda i: (0, i, 0)),
            pl.BlockSpec((NC, R, 16), lambda i: (0, i, 0)),
            pl.BlockSpec((R, D), lambda i: (i, 0)),
            pl.BlockSpec((D, D), lambda i: (0, 0)),
            pl.BlockSpec((1, D), lambda i: (0, 0)),
            pl.BlockSpec((D, D), lambda i: (0, 0)),
        ],
        out_specs=pl.BlockSpec((R, D), lambda i: (i, 0)),
        out_shape=jax.ShapeDtypeStruct((N, D), jnp.float32),
    )(sums, cnts, x, wlT, b, wrT)


def kernel(x, edge_index, W1_l, b1_l, W1_r, W2_l, b2_l, W2_r):
    src = edge_index[0].astype(jnp.int32).reshape(NW, NCH, K)
    dst = edge_index[1].astype(jnp.int32).reshape(NW, NCH, K)

    x_aug = jnp.concatenate(
        [x, jnp.ones((N, 1), jnp.float32), jnp.zeros((N, DA - D - 1),
                                                     jnp.float32)], axis=1)
    sums_aug = _make_sc_agg(DA)(x_aug, src, dst)
    sums1 = sums_aug[:, :, :D]
    cnts = sums_aug[:, :, D:]
    h = _tc_layer(True, sums1, cnts, x, W1_l.T, b1_l.reshape(1, D), W1_r.T)
    sums2 = _make_sc_agg(D)(h, src, dst)
    out = _tc_layer(False, sums2, cnts, h, W2_l.T, b2_l.reshape(1, D),
                    W2_r.T)
    return out


# speedup vs baseline: 7.6045x; 1.0690x over previous
import functools

import jax
import jax.numpy as jnp
from jax import lax
from jax.experimental import pallas as pl
from jax.experimental.pallas import tpu as pltpu
from jax.experimental.pallas import tpu_sc as plsc

N = 10000
NP = 10240
E = 320000
D = 128
DA = 144
NC = 2
NS = 16
NW = NC * NS
EPW = E // NW
K = 80
NCH = EPW // K
RPT = NP // NS

def _sc_agg_body(W, KC, x_hbm, src_hbm, dst_hbm, sum_out, src_v, dst_v,
                 rows0_v, rows1_v, sum_sp, sem):
    NCHC = EPW // KC
    c = lax.axis_index("c")
    s = lax.axis_index("s")
    wid = c * NS + s

    def _zrows(i, _):
        rows0_v[i // (W // 16), pl.ds((i % (W // 16)) * 16, 16)] = (
            jnp.zeros((16,), jnp.float32))
        return 0
    lax.fori_loop(0, KC * (W // 16), _zrows, 0)

    def _zsp(i, _):
        pltpu.sync_copy(rows0_v, sum_sp.at[pl.ds(s * RPT + i * KC, KC)])
        return 0
    lax.fori_loop(0, RPT // KC, _zsp, 0)

    pltpu.sync_copy(src_hbm.at[wid], src_v)
    pltpu.sync_copy(dst_hbm.at[wid], dst_v)

    plsc.subcore_barrier()

    def _gather(i, buf):
        pltpu.async_copy(x_hbm.at[src_v.at[i]], buf, sem)

    def _wait(buf):
        pltpu.make_async_copy(x_hbm.at[src_v.at[0]], buf, sem).wait()

    def _scat(i, buf):
        pltpu.sync_copy(buf, sum_sp.at[dst_v.at[i]], add=True)

    _gather(0, rows0_v)

    def _pair(j, _):
        i0 = 2 * j
        _wait(rows0_v)
        _gather(i0 + 1, rows1_v)
        _scat(i0, rows0_v)
        _wait(rows1_v)
        _gather(i0 + 2, rows0_v)
        _scat(i0 + 1, rows1_v)
        return 0

    if NCHC % 2 == 1:
        lax.fori_loop(0, (NCHC - 1) // 2, _pair, 0)
        _wait(rows0_v)
        _scat(NCHC - 1, rows0_v)
    else:
        lax.fori_loop(0, NCHC // 2 - 1, _pair, 0)
        _wait(rows0_v)
        _gather(NCHC - 1, rows1_v)
        _scat(NCHC - 2, rows0_v)
        _wait(rows1_v)
        _scat(NCHC - 1, rows1_v)

    plsc.subcore_barrier()

    pltpu.sync_copy(sum_sp.at[pl.ds(s * RPT, RPT)],
                    sum_out.at[c, pl.ds(s * RPT, RPT)])


@functools.cache
def _make_sc_agg(W, KC):
    mesh = plsc.VectorSubcoreMesh(core_axis_name="c", subcore_axis_name="s")
    return functools.partial(
        pl.kernel,
        mesh=mesh,
        out_type=jax.ShapeDtypeStruct((NC, NP, W), jnp.float32),
        scratch_types=[
            pltpu.VMEM((EPW // KC, KC), jnp.int32),
            pltpu.VMEM((EPW // KC, KC), jnp.int32),
            pltpu.VMEM((KC, W), jnp.float32),
            pltpu.VMEM((KC, W), jnp.float32),
            pltpu.VMEM_SHARED((NP, W), jnp.float32),
            pltpu.SemaphoreType.DMA,
        ],
        compiler_params=pltpu.CompilerParams(use_tc_tiling_on_sc=False),
    )(functools.partial(_sc_agg_body, W, KC))


R = 1000


def _tc_layer_body(relu, sum_ref, cnt_ref, x_ref, wlT_ref, b_ref, wrT_ref,
                   out_ref):
    ssum = sum_ref[0] + sum_ref[1]
    cnt = cnt_ref[0] + cnt_ref[1]
    inv = 1.0 / jnp.maximum(cnt[:, 0:1], 1.0)
    mean = ssum * inv
    acc = (jnp.dot(mean, wlT_ref[...], preferred_element_type=jnp.float32)
           + b_ref[...]
           + jnp.dot(x_ref[...], wrT_ref[...],
                     preferred_element_type=jnp.float32))
    out_ref[...] = jnp.maximum(acc, 0.0) if relu else acc


def _tc_layer(relu, sums, cnts, x, wlT, b, wrT):
    grid = (N // R,)
    return pl.pallas_call(
        functools.partial(_tc_layer_body, relu),
        grid=grid,
        in_specs=[
            pl.BlockSpec((NC, R, D), lambda i: (0, i, 0)),
            pl.BlockSpec((NC, R, 16), lambda i: (0, i, 0)),
            pl.BlockSpec((R, D), lambda i: (i, 0)),
            pl.BlockSpec((D, D), lambda i: (0, 0)),
            pl.BlockSpec((1, D), lambda i: (0, 0)),
            pl.BlockSpec((D, D), lambda i: (0, 0)),
        ],
        out_specs=pl.BlockSpec((R, D), lambda i: (i, 0)),
        out_shape=jax.ShapeDtypeStruct((N, D), jnp.float32),
    )(sums, cnts, x, wlT, b, wrT)


def kernel(x, edge_index, W1_l, b1_l, W1_r, W2_l, b2_l, W2_r):
    src = edge_index[0].astype(jnp.int32)
    dst = edge_index[1].astype(jnp.int32)

    x_aug = jnp.concatenate(
        [x, jnp.ones((N, 1), jnp.float32), jnp.zeros((N, DA - D - 1),
                                                     jnp.float32)], axis=1)
    k1 = 40
    sums_aug = _make_sc_agg(DA, k1)(x_aug,
                                    src.reshape(NW, EPW // k1, k1),
                                    dst.reshape(NW, EPW // k1, k1))
    sums1 = sums_aug[:, :, :D]
    cnts = sums_aug[:, :, D:]
    h = _tc_layer(True, sums1, cnts, x, W1_l.T, b1_l.reshape(1, D), W1_r.T)
    k2 = 80
    sums2 = _make_sc_agg(D, k2)(h, src.reshape(NW, EPW // k2, k2),
                                dst.reshape(NW, EPW // k2, k2))
    out = _tc_layer(False, sums2, cnts, h, W2_l.T, b2_l.reshape(1, D),
                    W2_r.T)
    return out


# speedup vs baseline: 9.2331x; 1.2142x over previous
import functools

import jax
import jax.numpy as jnp
from jax import lax
from jax.experimental import pallas as pl
from jax.experimental.pallas import tpu as pltpu
from jax.experimental.pallas import tpu_sc as plsc

N = 10000
NP = 10240
E = 320000
D = 128
DA = 144
NC = 2
NS = 16
NW = NC * NS
EPW = E // NW
K = 80
NCH = EPW // K
RPT = NP // NS

def _sc_agg_body(W, KC, NPH, x_hbm, src_hbm, dst_hbm, sum_out, src_v, dst_v,
                 rows0_v, rows1_v, sum_sp, sem):
    NCHC = EPW // KC
    base = NCHC // NPH
    sizes = [base + (1 if p < NCHC % NPH else 0) for p in range(NPH)]
    offs = [sum(sizes[:p]) for p in range(NPH)]
    c = lax.axis_index("c")
    s = lax.axis_index("s")
    wid = c * NS + s

    def _zrows(i, _):
        rows0_v[i // (W // 16), pl.ds((i % (W // 16)) * 16, 16)] = (
            jnp.zeros((16,), jnp.float32))
        return 0
    lax.fori_loop(0, KC * (W // 16), _zrows, 0)

    def _zsp(i, _):
        pltpu.sync_copy(rows0_v, sum_sp.at[pl.ds(s * RPT + i * KC, KC)])
        return 0
    lax.fori_loop(0, RPT // KC, _zsp, 0)

    plsc.subcore_barrier()

    def _gather(i, buf):
        pltpu.async_copy(x_hbm.at[src_v.at[i]], buf, sem)

    def _wait(buf):
        pltpu.make_async_copy(x_hbm.at[src_v.at[0]], buf, sem).wait()

    def _scat(i, buf):
        pltpu.sync_copy(buf, sum_sp.at[dst_v.at[i]], add=True)

    for ph, off in zip(sizes, offs):
        pltpu.sync_copy(src_hbm.at[wid, pl.ds(off, ph)],
                        src_v.at[pl.ds(0, ph)])
        pltpu.sync_copy(dst_hbm.at[wid, pl.ds(off, ph)],
                        dst_v.at[pl.ds(0, ph)])

        _gather(0, rows0_v)

        def _pair(j, _):
            i0 = 2 * j
            _wait(rows0_v)
            _gather(i0 + 1, rows1_v)
            _scat(i0, rows0_v)
            _wait(rows1_v)
            _gather(i0 + 2, rows0_v)
            _scat(i0 + 1, rows1_v)
            return 0

        if ph % 2 == 1:
            lax.fori_loop(0, (ph - 1) // 2, _pair, 0)
            _wait(rows0_v)
            _scat(ph - 1, rows0_v)
        else:
            lax.fori_loop(0, ph // 2 - 1, _pair, 0)
            _wait(rows0_v)
            _gather(ph - 1, rows1_v)
            _scat(ph - 2, rows0_v)
            _wait(rows1_v)
            _scat(ph - 1, rows1_v)

    plsc.subcore_barrier()

    pltpu.sync_copy(sum_sp.at[pl.ds(s * RPT, RPT)],
                    sum_out.at[c, pl.ds(s * RPT, RPT)])


@functools.cache
def _make_sc_agg(W, KC, NPH=1):
    NCHC = EPW // KC
    IB = NCHC // NPH + (1 if NCHC % NPH else 0)
    mesh = plsc.VectorSubcoreMesh(core_axis_name="c", subcore_axis_name="s")
    return functools.partial(
        pl.kernel,
        mesh=mesh,
        out_type=jax.ShapeDtypeStruct((NC, NP, W), jnp.float32),
        scratch_types=[
            pltpu.VMEM((IB, KC), jnp.int32),
            pltpu.VMEM((IB, KC), jnp.int32),
            pltpu.VMEM((KC, W), jnp.float32),
            pltpu.VMEM((KC, W), jnp.float32),
            pltpu.VMEM_SHARED((NP, W), jnp.float32),
            pltpu.SemaphoreType.DMA,
        ],
        compiler_params=pltpu.CompilerParams(use_tc_tiling_on_sc=False),
    )(functools.partial(_sc_agg_body, W, KC, NPH))


R = 1000


def _tc_layer_body(relu, WS, sum_ref, x_ref, wl_ref, b_ref, wr_ref,
                   out_ref):
    ssum = sum_ref[0, :, :D] + sum_ref[1, :, :D]
    if WS > D:
        cnt = sum_ref[0, :, D:D + 1] + sum_ref[1, :, D:D + 1]
    else:
        cnt = b_ref[0, 0:1]
    inv = 1.0 / jnp.maximum(cnt, 1.0)
    mean = ssum * inv
    dn = (((1,), (1,)), ((), ()))
    acc = (lax.dot_general(mean, wl_ref[...], dn,
                           preferred_element_type=jnp.float32)
           + b_ref[...]
           + lax.dot_general(x_ref[...], wr_ref[...], dn,
                             preferred_element_type=jnp.float32))
    out_ref[...] = jnp.maximum(acc, 0.0) if relu else acc


def _tc_layer(relu, sums, cnt_sums, x, wl, b, wr):
    WS = sums.shape[-1]
    if WS > D:
        ins = (sums, x, wl, b.reshape(1, D), wr)
        sspec = pl.BlockSpec((NC, R, WS), lambda i: (0, i, 0))
        body = functools.partial(_tc_layer_body, relu, WS)
        specs = [sspec,
                 pl.BlockSpec((R, D), lambda i: (i, 0)),
                 pl.BlockSpec((D, D), lambda i: (0, 0)),
                 pl.BlockSpec((1, D), lambda i: (0, 0)),
                 pl.BlockSpec((D, D), lambda i: (0, 0))]
    else:
        ins = (sums, cnt_sums, x, wl, b.reshape(1, D), wr)
        body = functools.partial(_tc_layer2_body, relu)
        specs = [pl.BlockSpec((NC, R, D), lambda i: (0, i, 0)),
                 pl.BlockSpec((NC, R, 16), lambda i: (0, i, 0)),
                 pl.BlockSpec((R, D), lambda i: (i, 0)),
                 pl.BlockSpec((D, D), lambda i: (0, 0)),
                 pl.BlockSpec((1, D), lambda i: (0, 0)),
                 pl.BlockSpec((D, D), lambda i: (0, 0))]
    return pl.pallas_call(
        body,
        grid=(N // R,),
        in_specs=specs,
        out_specs=pl.BlockSpec((R, D), lambda i: (i, 0)),
        out_shape=jax.ShapeDtypeStruct((N, D), jnp.float32),
    )(*ins)


def _tc_layer2_body(relu, sum_ref, cnt_ref, x_ref, wl_ref, b_ref, wr_ref,
                    out_ref):
    ssum = sum_ref[0] + sum_ref[1]
    cnt = cnt_ref[0, :, 0:1] + cnt_ref[1, :, 0:1]
    inv = 1.0 / jnp.maximum(cnt, 1.0)
    mean = ssum * inv
    dn = (((1,), (1,)), ((), ()))
    acc = (lax.dot_general(mean, wl_ref[...], dn,
                           preferred_element_type=jnp.float32)
           + b_ref[...]
           + lax.dot_general(x_ref[...], wr_ref[...], dn,
                             preferred_element_type=jnp.float32))
    out_ref[...] = jnp.maximum(acc, 0.0) if relu else acc


def kernel(x, edge_index, W1_l, b1_l, W1_r, W2_l, b2_l, W2_r):
    src = edge_index[0].astype(jnp.int32)
    dst = edge_index[1].astype(jnp.int32)

    x_aug = jnp.concatenate(
        [x, jnp.ones((N, 1), jnp.float32), jnp.zeros((N, DA - D - 1),
                                                     jnp.float32)], axis=1)
    src3 = src.reshape(NW, EPW // 80, 80)
    dst3 = dst.reshape(NW, EPW // 80, 80)
    sums_aug = _make_sc_agg(DA, 80, 2)(x_aug, src3, dst3)
    h = _tc_layer(True, sums_aug, None, x, W1_l, b1_l, W1_r)
    sums2 = _make_sc_agg(D, 80, 1)(h, src3, dst3)
    cnts = lax.slice(sums_aug, (0, 0, D), (NC, NP, DA))
    out = _tc_layer(False, sums2, cnts, h, W2_l, b2_l, W2_r)
    return out


# speedup vs baseline: 14.6426x; 1.5859x over previous
import functools

import jax
import jax.numpy as jnp
from jax import lax
from jax.experimental import pallas as pl
from jax.experimental.pallas import tpu as pltpu
from jax.experimental.pallas import tpu_sc as plsc

N = 10000
NP = 10240
E = 320000
D = 128
DA = 144
NC = 2
NS = 16
NW = NC * NS
EPW = E // NW
K = 80
NCH = EPW // K
RPT = NP // NS

def _sc_agg_body(W, KC, NPH, x_hbm, e_hbm, sum_out, src_v, dst_v,
                 rows0_v, rows1_v, rows2_v, sum_sp, gs0):
    NCHC = EPW // KC
    base = NCHC // NPH
    sizes = [base + (1 if p < NCHC % NPH else 0) for p in range(NPH)]
    offs = [sum(sizes[:p]) for p in range(NPH)]
    c = lax.axis_index("c")
    s = lax.axis_index("s")
    wid = c * NS + s

    def _zrows(i, _):
        rows0_v[i // (W // 16), pl.ds((i % (W // 16)) * 16, 16)] = (
            jnp.zeros((16,), jnp.float32))
        return 0
    lax.fori_loop(0, KC * (W // 16), _zrows, 0)

    def _zsp(i, _):
        pltpu.sync_copy(rows0_v, sum_sp.at[pl.ds(s * RPT + i * KC, KC)])
        return 0
    lax.fori_loop(0, RPT // KC, _zsp, 0)

    plsc.subcore_barrier()

    def _gather(i, buf):
        pltpu.async_copy(x_hbm.at[src_v.at[i]], buf, gs0)

    def _wait(buf):
        pltpu.make_async_copy(x_hbm.at[src_v.at[0]], buf, gs0).wait()

    def _scat(i, buf):
        pltpu.sync_copy(buf, sum_sp.at[dst_v.at[i]], add=True)

    for ph, off in zip(sizes, offs):
        pltpu.sync_copy(e_hbm.at[0, wid, pl.ds(off, ph)],
                        src_v.at[pl.ds(0, ph)])
        pltpu.sync_copy(e_hbm.at[1, wid, pl.ds(off, ph)],
                        dst_v.at[pl.ds(0, ph)])

        bufs = (rows0_v, rows1_v, rows2_v)
        _gather(0, rows0_v)
        _gather(1, rows1_v)
        _gather(2, rows2_v)

        def _triple(j, _):
            i0 = 3 * j
            for b in range(3):
                _wait(bufs[b])
                _scat(i0 + b, bufs[b])
                _gather(i0 + b + 3, bufs[b])
            return 0

        M = (ph - 3) // 3
        lax.fori_loop(0, M, _triple, 0)
        t0 = 3 * M
        rem = ph - t0
        for b in range(3):
            _wait(bufs[b])
            _scat(t0 + b, bufs[b])
            if rem > 3 + b:
                _gather(t0 + b + 3, bufs[b])
        if rem > 3:
            _wait(bufs[0])
            _scat(t0 + 3, bufs[0])
        if rem > 4:
            _wait(bufs[1])
            _scat(t0 + 4, bufs[1])

    plsc.subcore_barrier()

    pltpu.sync_copy(sum_sp.at[pl.ds(s * RPT, RPT)],
                    sum_out.at[c, pl.ds(s * RPT, RPT)])


def _sc_agg_cnt_body(KC, NPH, x_hbm, e_hbm, sum_out, cnt_out,
                     src_v, dst_v, rows0_v, rows1_v, rows2_v, ones_v, sum_sp,
                     cnt_sp, gs0):
    W = D
    NCHC = EPW // KC
    base = NCHC // NPH
    sizes = [base + (1 if p < NCHC % NPH else 0) for p in range(NPH)]
    offs = [sum(sizes[:p]) for p in range(NPH)]
    c = lax.axis_index("c")
    s = lax.axis_index("s")
    wid = c * NS + s

    def _zrows(i, _):
        rows0_v[i // (W // 16), pl.ds((i % (W // 16)) * 16, 16)] = (
            jnp.zeros((16,), jnp.float32))
        return 0
    lax.fori_loop(0, KC * (W // 16), _zrows, 0)

    def _zones(v):
        def f(i, _):
            ones_v[i, pl.ds(0, 16)] = jnp.full((16,), v, jnp.float32)
            return 0
        lax.fori_loop(0, KC, f, 0)

    _zones(0.0)

    def _zsp(i, _):
        pltpu.sync_copy(rows0_v, sum_sp.at[pl.ds(s * RPT + i * KC, KC)])
        pltpu.sync_copy(ones_v, cnt_sp.at[pl.ds(s * RPT + i * KC, KC)])
        return 0
    lax.fori_loop(0, RPT // KC, _zsp, 0)

    _zones(1.0)

    plsc.subcore_barrier()

    def _gather(i, buf):
        pltpu.async_copy(x_hbm.at[src_v.at[i]], buf, gs0)

    def _wait(buf):
        pltpu.make_async_copy(x_hbm.at[src_v.at[0]], buf, gs0).wait()

    def _scat(i, buf):
        pltpu.sync_copy(buf, sum_sp.at[dst_v.at[i]], add=True)
        pltpu.sync_copy(ones_v, cnt_sp.at[dst_v.at[i]], add=True)

    for ph, off in zip(sizes, offs):
        pltpu.sync_copy(e_hbm.at[0, wid, pl.ds(off, ph)],
                        src_v.at[pl.ds(0, ph)])
        pltpu.sync_copy(e_hbm.at[1, wid, pl.ds(off, ph)],
                        dst_v.at[pl.ds(0, ph)])

        bufs = (rows0_v, rows1_v, rows2_v)
        _gather(0, rows0_v)
        _gather(1, rows1_v)
        _gather(2, rows2_v)

        def _triple(j, _):
            i0 = 3 * j
            for b in range(3):
                _wait(bufs[b])
                _scat(i0 + b, bufs[b])
                _gather(i0 + b + 3, bufs[b])
            return 0

        M = (ph - 3) // 3
        lax.fori_loop(0, M, _triple, 0)
        t0 = 3 * M
        rem = ph - t0
        for b in range(3):
            _wait(bufs[b])
            _scat(t0 + b, bufs[b])
            if rem > 3 + b:
                _gather(t0 + b + 3, bufs[b])
        if rem > 3:
            _wait(bufs[0])
            _scat(t0 + 3, bufs[0])
        if rem > 4:
            _wait(bufs[1])
            _scat(t0 + 4, bufs[1])

    plsc.subcore_barrier()

    pltpu.sync_copy(sum_sp.at[pl.ds(s * RPT, RPT)],
                    sum_out.at[c, pl.ds(s * RPT, RPT)])
    pltpu.sync_copy(cnt_sp.at[pl.ds(s * RPT, RPT)],
                    cnt_out.at[c, pl.ds(s * RPT, RPT)])


@functools.cache
def _make_sc_agg_cnt(KC=80, NPH=2):
    NCHC = EPW // KC
    IB = NCHC // NPH + (1 if NCHC % NPH else 0)
    mesh = plsc.VectorSubcoreMesh(core_axis_name="c", subcore_axis_name="s")
    return functools.partial(
        pl.kernel,
        mesh=mesh,
        out_type=(
            jax.ShapeDtypeStruct((NC, NP, D), jnp.float32),
            jax.ShapeDtypeStruct((NC, NP, 16), jnp.float32),
        ),
        scratch_types=[
            pltpu.VMEM((IB, KC), jnp.int32),
            pltpu.VMEM((IB, KC), jnp.int32),
            pltpu.VMEM((KC, D), jnp.float32),
            pltpu.VMEM((KC, D), jnp.float32),
            pltpu.VMEM((KC, D), jnp.float32),
            pltpu.VMEM((KC, 16), jnp.float32),
            pltpu.VMEM_SHARED((NP, D), jnp.float32),
            pltpu.VMEM_SHARED((NP, 16), jnp.float32),
            pltpu.SemaphoreType.DMA,
        ],
        compiler_params=pltpu.CompilerParams(use_tc_tiling_on_sc=False),
    )(functools.partial(_sc_agg_cnt_body, KC, NPH))


@functools.cache
def _make_sc_agg(W, KC, NPH=1):
    NCHC = EPW // KC
    IB = NCHC // NPH + (1 if NCHC % NPH else 0)
    mesh = plsc.VectorSubcoreMesh(core_axis_name="c", subcore_axis_name="s")
    return functools.partial(
        pl.kernel,
        mesh=mesh,
        out_type=jax.ShapeDtypeStruct((NC, NP, W), jnp.float32),
        scratch_types=[
            pltpu.VMEM((IB, KC), jnp.int32),
            pltpu.VMEM((IB, KC), jnp.int32),
            pltpu.VMEM((KC, W), jnp.float32),
            pltpu.VMEM((KC, W), jnp.float32),
            pltpu.VMEM((KC, W), jnp.float32),
            pltpu.VMEM_SHARED((NP, W), jnp.float32),
            pltpu.SemaphoreType.DMA,
        ],
        compiler_params=pltpu.CompilerParams(use_tc_tiling_on_sc=False),
    )(functools.partial(_sc_agg_body, W, KC, NPH))


R = 1000


def _tc_layer_body(relu, WS, sum_ref, x_ref, wl_ref, b_ref, wr_ref,
                   out_ref):
    ssum = sum_ref[0, :, :D] + sum_ref[1, :, :D]
    if WS > D:
        cnt = sum_ref[0, :, D:D + 1] + sum_ref[1, :, D:D + 1]
    else:
        cnt = b_ref[0, 0:1]
    inv = 1.0 / jnp.maximum(cnt, 1.0)
    mean = ssum * inv
    dn = (((1,), (1,)), ((), ()))
    acc = (lax.dot_general(mean, wl_ref[...], dn,
                           preferred_element_type=jnp.float32)
           + b_ref[...]
           + lax.dot_general(x_ref[...], wr_ref[...], dn,
                             preferred_element_type=jnp.float32))
    out_ref[...] = jnp.maximum(acc, 0.0) if relu else acc


def _tc_layer(relu, sums, cnt_sums, x, wl, b, wr):
    WS = sums.shape[-1]
    if WS > D:
        ins = (sums, x, wl, b.reshape(1, D), wr)
        sspec = pl.BlockSpec((NC, R, WS), lambda i: (0, i, 0))
        body = functools.partial(_tc_layer_body, relu, WS)
        specs = [sspec,
                 pl.BlockSpec((R, D), lambda i: (i, 0)),
                 pl.BlockSpec((D, D), lambda i: (0, 0)),
                 pl.BlockSpec((1, D), lambda i: (0, 0)),
                 pl.BlockSpec((D, D), lambda i: (0, 0))]
    else:
        ins = (sums, cnt_sums, x, wl, b.reshape(1, D), wr)
        body = functools.partial(_tc_layer2_body, relu)
        specs = [pl.BlockSpec((NC, R, D), lambda i: (0, i, 0)),
                 pl.BlockSpec((NC, R, 16), lambda i: (0, i, 0)),
                 pl.BlockSpec((R, D), lambda i: (i, 0)),
                 pl.BlockSpec((D, D), lambda i: (0, 0)),
                 pl.BlockSpec((1, D), lambda i: (0, 0)),
                 pl.BlockSpec((D, D), lambda i: (0, 0))]
    return pl.pallas_call(
        body,
        grid=(N // R,),
        in_specs=specs,
        out_specs=pl.BlockSpec((R, D), lambda i: (i, 0)),
        out_shape=jax.ShapeDtypeStruct((N, D), jnp.float32),
    )(*ins)


def _tc_layer2_body(relu, sum_ref, cnt_ref, x_ref, wl_ref, b_ref, wr_ref,
                    out_ref):
    ssum = sum_ref[0] + sum_ref[1]
    cnt = cnt_ref[0, :, 0:1] + cnt_ref[1, :, 0:1]
    inv = 1.0 / jnp.maximum(cnt, 1.0)
    mean = ssum * inv
    dn = (((1,), (1,)), ((), ()))
    acc = (lax.dot_general(mean, wl_ref[...], dn,
                           preferred_element_type=jnp.float32)
           + b_ref[...]
           + lax.dot_general(x_ref[...], wr_ref[...], dn,
                             preferred_element_type=jnp.float32))
    out_ref[...] = jnp.maximum(acc, 0.0) if relu else acc


def kernel(x, edge_index, W1_l, b1_l, W1_r, W2_l, b2_l, W2_r):
    e4 = edge_index.astype(jnp.int32).reshape(2, NW, EPW // 80, 80)

    sums1, cnts = _make_sc_agg_cnt(80, 4)(x, e4)
    h = _tc_layer(True, sums1, cnts, x, W1_l, b1_l, W1_r)
    sums2 = _make_sc_agg(D, 80, 2)(h, e4)
    out = _tc_layer(False, sums2, cnts, h, W2_l, b2_l, W2_r)
    return out
